# baseline (speedup 1.0000x reference)
import functools

import jax
import jax.numpy as jnp
from jax import lax
from jax.experimental import pallas as pl
from jax.experimental.pallas import tpu as pltpu
from jax.experimental.pallas import tpu_sc as plsc

NC = 2
NS = 16
NW = NC * NS
CHUNK = 128


@functools.lru_cache(maxsize=None)
def _make_gather(B: int, D: int):
    rows_per_w = B // NW
    nchunk = rows_per_w // CHUNK
    mesh = plsc.VectorSubcoreMesh(
        core_axis_name="c", subcore_axis_name="s",
        num_cores=NC, num_subcores=NS)

    @functools.partial(
        pl.kernel,
        out_type=(jax.ShapeDtypeStruct((B, D), jnp.float32),
                  jax.ShapeDtypeStruct((B, D), jnp.float32)),
        mesh=mesh,
        scratch_types=[
            pltpu.VMEM((nchunk, CHUNK), jnp.int32),
            pltpu.VMEM((nchunk, CHUNK), jnp.int32),
            pltpu.VMEM((CHUNK, D), jnp.float32),
            pltpu.SemaphoreType.DMA,
        ],
        compiler_params=pltpu.CompilerParams(use_tc_tiling_on_sc=False),
    )
    def gather(u_idx_hbm, i_idx_hbm, u_emb_hbm, i_emb_hbm,
               ue_hbm, ie_hbm, idx_u_v, idx_i_v, rows_v, sem):
        wid = lax.axis_index("s") * NC + lax.axis_index("c")
        crow = wid * nchunk
        pltpu.sync_copy(u_idx_hbm.at[pl.ds(crow, nchunk)], idx_u_v)
        pltpu.sync_copy(i_idx_hbm.at[pl.ds(crow, nchunk)], idx_i_v)
        base = wid * rows_per_w
        for j in range(nchunk):
            pltpu.async_copy(u_emb_hbm.at[idx_u_v.at[j]], rows_v, sem).wait()
            pltpu.sync_copy(rows_v, ue_hbm.at[pl.ds(base + j * CHUNK, CHUNK)])
        for j in range(nchunk):
            pltpu.async_copy(i_emb_hbm.at[idx_i_v.at[j]], rows_v, sem).wait()
            pltpu.sync_copy(rows_v, ie_hbm.at[pl.ds(base + j * CHUNK, CHUNK)])

    return gather


@functools.lru_cache(maxsize=None)
def _make_mlp(B: int, D: int, H: int, blk: int):
    def mlp(ue_ref, ie_ref, w1u_ref, w1i_ref, b1_ref, w2_ref, b2_ref, out_ref):
        h = jnp.dot(ue_ref[...], w1u_ref[...], preferred_element_type=jnp.float32)
        h = h + jnp.dot(ie_ref[...], w1i_ref[...], preferred_element_type=jnp.float32)
        h = jnp.maximum(h + b1_ref[...], 0.0)
        z = jnp.sum(h * w2_ref[...], axis=1) + b2_ref[0]
        out_ref[...] = jax.nn.sigmoid(z)

    return pl.pallas_call(
        mlp,
        grid=(B // blk,),
        in_specs=[
            pl.BlockSpec((blk, D), lambda b: (b, 0)),
            pl.BlockSpec((blk, D), lambda b: (b, 0)),
            pl.BlockSpec((D, H), lambda b: (0, 0)),
            pl.BlockSpec((D, H), lambda b: (0, 0)),
            pl.BlockSpec((1, H), lambda b: (0, 0)),
            pl.BlockSpec((1, H), lambda b: (0, 0)),
            pl.BlockSpec(memory_space=pltpu.SMEM),
        ],
        out_specs=pl.BlockSpec((blk,), lambda b: (b,)),
        out_shape=jax.ShapeDtypeStruct((B,), jnp.float32),
    )


def kernel(u, i, u_emb, i_emb, W1, b1, W2, b2):
    B = u.shape[0]
    _, D = u_emb.shape
    H = W1.shape[0]
    u_idx = u.astype(jnp.int32).reshape(B // CHUNK, CHUNK)
    i_idx = i.astype(jnp.int32).reshape(B // CHUNK, CHUNK)
    ue, ie = _make_gather(B, D)(u_idx, i_idx, u_emb, i_emb)
    w1u = W1[:, :D].T
    w1i = W1[:, D:].T
    return _make_mlp(B, D, H, 2048)(
        ue, ie, w1u, w1i, b1.reshape(1, H), W2, b2)


# speedup vs baseline: 2.1987x; 2.1987x over previous
import functools

import jax
import jax.numpy as jnp
from jax import lax
from jax.experimental import pallas as pl
from jax.experimental.pallas import tpu as pltpu
from jax.experimental.pallas import tpu_sc as plsc

NC = 2
NS = 16
NW = NC * NS
LT = 128
RING = 2


@functools.lru_cache(maxsize=None)
def _make_gather(B: int, D: int, V: int):
    rows_per_w = B // NW
    nvec = rows_per_w // 16
    mesh = plsc.VectorSubcoreMesh(
        core_axis_name="c", subcore_axis_name="s",
        num_cores=NC, num_subcores=NS)

    grp = [pltpu.VMEM((D, LT), jnp.float32)] * (2 * RING)
    sems = [pltpu.SemaphoreType.DMA] * (2 * RING)

    @functools.partial(
        pl.kernel,
        out_type=(jax.ShapeDtypeStruct((B, D), jnp.float32),
                  jax.ShapeDtypeStruct((B, D), jnp.float32)),
        mesh=mesh,
        scratch_types=[
            pltpu.VMEM((nvec, 16), jnp.int32),
            pltpu.VMEM((nvec, 16), jnp.int32),
            pltpu.VMEM((rows_per_w // 2, D), jnp.float32),
            pltpu.VMEM((rows_per_w // 2, D), jnp.float32),
            *grp,
            *sems,
        ],
        compiler_params=pltpu.CompilerParams(needs_layout_passes=False),
    )
    def gather(u_idx_hbm, i_idx_hbm, u_embT_hbm, i_embT_hbm, ue_hbm, ie_hbm,
               idx_u_v, idx_i_v, rows_u_v, rows_i_v, *ring):
        bufs = ring[:2 * RING]
        sems = ring[2 * RING:]
        wid = lax.axis_index("s") * NC + lax.axis_index("c")
        base = wid * rows_per_w
        pltpu.sync_copy(u_idx_hbm.at[pl.ds(wid * nvec, nvec)], idx_u_v)
        pltpu.sync_copy(i_idx_hbm.at[pl.ds(wid * nvec, nvec)], idx_i_v)

        iotas = [lax.iota(jnp.int32, 16) + 16 * c for c in range(D // 16)]

        def fire(tbl, emb_hbm, v, l):
            g0 = pl.multiple_of((v[l] // LT) * LT, LT)
            slot = tbl * RING + (l % RING)
            pltpu.make_async_copy(
                emb_hbm.at[:, pl.ds(g0, LT)], bufs[slot], sems[slot]).start()

        def extract(tbl, emb_hbm, rows_v, v, l, row):
            slot = tbl * RING + (l % RING)
            pltpu.make_async_copy(
                emb_hbm.at[:, pl.ds(0, LT)], bufs[slot], sems[slot]).wait()
            lane = jnp.broadcast_to(v[l] % LT, (16,))
            dst = rows_v.at[row]
            for c in range(D // 16):
                dst[pl.ds(16 * c, 16)] = plsc.load_gather(
                    bufs[slot], [iotas[c], lane])

        hvec = nvec // 2
        for half in range(2):
            def body(k, carry):
                pu, pi = carry
                vu = idx_u_v[half * hvec + k]
                vi = idx_i_v[half * hvec + k]
                for l in range(16):
                    e = k * 16 + l - RING
                    if l >= RING:
                        extract(0, u_embT_hbm, rows_u_v, vu, l - RING, e)
                        extract(1, i_embT_hbm, rows_i_v, vi, l - RING, e)
                    else:
                        @pl.when(k > 0)
                        def _():
                            extract(0, u_embT_hbm, rows_u_v, pu, l - RING, e)
                            extract(1, i_embT_hbm, rows_i_v, pi, l - RING, e)
                    fire(0, u_embT_hbm, vu, l)
                    fire(1, i_embT_hbm, vi, l)
                return vu, vi

            last_u, last_i = lax.fori_loop(
                0, hvec, body,
                (idx_u_v[half * hvec], idx_i_v[half * hvec]))
            for l in range(16 - RING, 16):
                e = (hvec - 1) * 16 + l
                extract(0, u_embT_hbm, rows_u_v, last_u, l, e)
                extract(1, i_embT_hbm, rows_i_v, last_i, l, e)

            out0 = base + half * (rows_per_w // 2)
            pltpu.sync_copy(rows_u_v, ue_hbm.at[pl.ds(out0, rows_per_w // 2)])
            pltpu.sync_copy(rows_i_v, ie_hbm.at[pl.ds(out0, rows_per_w // 2)])

    return gather


@functools.lru_cache(maxsize=None)
def _make_mlp(B: int, D: int, H: int, blk: int):
    def mlp(ue_ref, ie_ref, w1u_ref, w1i_ref, b1_ref, w2_ref, b2_ref, out_ref):
        h = jnp.dot(ue_ref[...], w1u_ref[...], preferred_element_type=jnp.float32)
        h = h + jnp.dot(ie_ref[...], w1i_ref[...], preferred_element_type=jnp.float32)
        h = jnp.maximum(h + b1_ref[...], 0.0)
        z = jnp.sum(h * w2_ref[...], axis=1) + b2_ref[0]
        out_ref[...] = jax.nn.sigmoid(z)

    return pl.pallas_call(
        mlp,
        grid=(B // blk,),
        in_specs=[
            pl.BlockSpec((blk, D), lambda b: (b, 0)),
            pl.BlockSpec((blk, D), lambda b: (b, 0)),
            pl.BlockSpec((D, H), lambda b: (0, 0)),
            pl.BlockSpec((D, H), lambda b: (0, 0)),
            pl.BlockSpec((1, H), lambda b: (0, 0)),
            pl.BlockSpec((1, H), lambda b: (0, 0)),
            pl.BlockSpec(memory_space=pltpu.SMEM),
        ],
        out_specs=pl.BlockSpec((blk,), lambda b: (b,)),
        out_shape=jax.ShapeDtypeStruct((B,), jnp.float32),
    )


def kernel(u, i, u_emb, i_emb, W1, b1, W2, b2):
    B = u.shape[0]
    V, D = u_emb.shape
    H = W1.shape[0]
    u_embT = jnp.swapaxes(u_emb, 0, 1)
    i_embT = jnp.swapaxes(i_emb, 0, 1)
    ue, ie = _make_gather(B, D, V)(
        u.astype(jnp.int32).reshape(B // 16, 16),
        i.astype(jnp.int32).reshape(B // 16, 16), u_embT, i_embT)
    w1u = W1[:, :D].T
    w1i = W1[:, D:].T
    return _make_mlp(B, D, H, 2048)(
        ue, ie, w1u, w1i, b1.reshape(1, H), W2, b2)


# speedup vs baseline: 2.4247x; 1.1028x over previous
import functools

import jax
import jax.numpy as jnp
from jax import lax
from jax.experimental import pallas as pl
from jax.experimental.pallas import tpu as pltpu
from jax.experimental.pallas import tpu_sc as plsc

NC = 2
NS = 16
NW = NC * NS
LT = 128
RING = 3


@functools.lru_cache(maxsize=None)
def _make_gather(B: int, D: int, V: int):
    rows_per_w = B // NW
    nvec = rows_per_w // 16
    mesh = plsc.VectorSubcoreMesh(
        core_axis_name="c", subcore_axis_name="s",
        num_cores=NC, num_subcores=NS)

    grp = [pltpu.VMEM((D, LT), jnp.float32)] * (2 * RING)
    sems = [pltpu.SemaphoreType.DMA] * (2 * RING)

    @functools.partial(
        pl.kernel,
        out_type=(jax.ShapeDtypeStruct((B, D), jnp.float32),
                  jax.ShapeDtypeStruct((B, D), jnp.float32)),
        mesh=mesh,
        scratch_types=[
            pltpu.VMEM((nvec, 16), jnp.int32),
            pltpu.VMEM((nvec, 16), jnp.int32),
            pltpu.VMEM((rows_per_w // 2, D), jnp.float32),
            pltpu.VMEM((rows_per_w // 2, D), jnp.float32),
            *grp,
            *sems,
        ],
        compiler_params=pltpu.CompilerParams(needs_layout_passes=False),
    )
    def gather(u_idx_hbm, i_idx_hbm, u_embT_hbm, i_embT_hbm, ue_hbm, ie_hbm,
               idx_u_v, idx_i_v, rows_u_v, rows_i_v, *ring):
        bufs = ring[:2 * RING]
        sems = ring[2 * RING:]
        wid = lax.axis_index("s") * NC + lax.axis_index("c")
        base = wid * rows_per_w
        pltpu.sync_copy(u_idx_hbm.at[pl.ds(wid * nvec, nvec)], idx_u_v)
        pltpu.sync_copy(i_idx_hbm.at[pl.ds(wid * nvec, nvec)], idx_i_v)

        iotas = [lax.iota(jnp.int32, 16) + 16 * c for c in range(D // 16)]

        def fire(tbl, emb_hbm, v, l):
            g0 = pl.multiple_of((v[l] // LT) * LT, LT)
            slot = tbl * RING + (l % RING)
            pltpu.make_async_copy(
                emb_hbm.at[:, pl.ds(g0, LT)], bufs[slot], sems[slot]).start()

        def extract(tbl, emb_hbm, rows_v, v, l, row):
            slot = tbl * RING + (l % RING)
            pltpu.make_async_copy(
                emb_hbm.at[:, pl.ds(0, LT)], bufs[slot], sems[slot]).wait()
            lane = jnp.broadcast_to(v[l] % LT, (16,))
            dst = rows_v.at[row]
            for c in range(D // 16):
                dst[pl.ds(16 * c, 16)] = plsc.load_gather(
                    bufs[slot], [iotas[c], lane])

        hvec = nvec // 2
        for half in range(2):
            def body(k, carry):
                pu, pi = carry
                vu = idx_u_v[half * hvec + k]
                vi = idx_i_v[half * hvec + k]
                for l in range(16):
                    e = k * 16 + l - RING
                    if l >= RING:
                        extract(0, u_embT_hbm, rows_u_v, vu, l - RING, e)
                        extract(1, i_embT_hbm, rows_i_v, vi, l - RING, e)
                    else:
                        @pl.when(k > 0)
                        def _():
                            extract(0, u_embT_hbm, rows_u_v, pu, l - RING, e)
                            extract(1, i_embT_hbm, rows_i_v, pi, l - RING, e)
                    fire(0, u_embT_hbm, vu, l)
                    fire(1, i_embT_hbm, vi, l)
                return vu, vi

            last_u, last_i = lax.fori_loop(
                0, hvec, body,
                (idx_u_v[half * hvec], idx_i_v[half * hvec]))
            for l in range(16 - RING, 16):
                e = (hvec - 1) * 16 + l
                extract(0, u_embT_hbm, rows_u_v, last_u, l, e)
                extract(1, i_embT_hbm, rows_i_v, last_i, l, e)

            out0 = base + half * (rows_per_w // 2)
            pltpu.sync_copy(rows_u_v, ue_hbm.at[pl.ds(out0, rows_per_w // 2)])
            pltpu.sync_copy(rows_i_v, ie_hbm.at[pl.ds(out0, rows_per_w // 2)])

    return gather


@functools.lru_cache(maxsize=None)
def _make_mlp(B: int, D: int, H: int, blk: int):
    def mlp(ue_ref, ie_ref, w1u_ref, w1i_ref, b1_ref, w2_ref, b2_ref, out_ref):
        h = jnp.dot(ue_ref[...], w1u_ref[...], preferred_element_type=jnp.float32)
        h = h + jnp.dot(ie_ref[...], w1i_ref[...], preferred_element_type=jnp.float32)
        h = jnp.maximum(h + b1_ref[...], 0.0)
        z = jnp.sum(h * w2_ref[...], axis=1) + b2_ref[0]
        out_ref[...] = jax.nn.sigmoid(z)

    return pl.pallas_call(
        mlp,
        grid=(B // blk,),
        in_specs=[
            pl.BlockSpec((blk, D), lambda b: (b, 0)),
            pl.BlockSpec((blk, D), lambda b: (b, 0)),
            pl.BlockSpec((D, H), lambda b: (0, 0)),
            pl.BlockSpec((D, H), lambda b: (0, 0)),
            pl.BlockSpec((1, H), lambda b: (0, 0)),
            pl.BlockSpec((1, H), lambda b: (0, 0)),
            pl.BlockSpec(memory_space=pltpu.SMEM),
        ],
        out_specs=pl.BlockSpec((blk,), lambda b: (b,)),
        out_shape=jax.ShapeDtypeStruct((B,), jnp.float32),
    )


def kernel(u, i, u_emb, i_emb, W1, b1, W2, b2):
    B = u.shape[0]
    V, D = u_emb.shape
    H = W1.shape[0]
    u_embT = jnp.swapaxes(u_emb, 0, 1)
    i_embT = jnp.swapaxes(i_emb, 0, 1)
    ue, ie = _make_gather(B, D, V)(
        u.astype(jnp.int32).reshape(B // 16, 16),
        i.astype(jnp.int32).reshape(B // 16, 16), u_embT, i_embT)
    w1u = W1[:, :D].T
    w1i = W1[:, D:].T
    return _make_mlp(B, D, H, 2048)(
        ue, ie, w1u, w1i, b1.reshape(1, H), W2, b2)


# speedup vs baseline: 2.6394x; 1.0886x over previous
import functools

import jax
import jax.numpy as jnp
from jax import lax
from jax.experimental import pallas as pl
from jax.experimental.pallas import tpu as pltpu
from jax.experimental.pallas import tpu_sc as plsc

NC = 2
NS = 16
NW = NC * NS
LT = 128
RING = 4


@functools.lru_cache(maxsize=None)
def _make_gather(B: int, D: int, V: int):
    rows_per_w = B // NW
    nvec = rows_per_w // 16
    mesh = plsc.VectorSubcoreMesh(
        core_axis_name="c", subcore_axis_name="s",
        num_cores=NC, num_subcores=NS)

    grp = [pltpu.VMEM((D, LT), jnp.float32)] * (2 * RING)
    sems = [pltpu.SemaphoreType.DMA] * (2 * RING)

    @functools.partial(
        pl.kernel,
        out_type=(jax.ShapeDtypeStruct((B, D), jnp.float32),
                  jax.ShapeDtypeStruct((B, D), jnp.float32)),
        mesh=mesh,
        scratch_types=[
            pltpu.VMEM((nvec, 16), jnp.int32),
            pltpu.VMEM((nvec, 16), jnp.int32),
            pltpu.VMEM((rows_per_w // 4, D), jnp.float32),
            pltpu.VMEM((rows_per_w // 4, D), jnp.float32),
            *grp,
            *sems,
        ],
        compiler_params=pltpu.CompilerParams(needs_layout_passes=False),
    )
    def gather(u_idx_hbm, i_idx_hbm, u_embT_hbm, i_embT_hbm, ue_hbm, ie_hbm,
               idx_u_v, idx_i_v, rows_u_v, rows_i_v, *ring):
        bufs = ring[:2 * RING]
        sems = ring[2 * RING:]
        wid = lax.axis_index("s") * NC + lax.axis_index("c")
        base = wid * rows_per_w
        pltpu.sync_copy(u_idx_hbm.at[pl.ds(wid * nvec, nvec)], idx_u_v)
        pltpu.sync_copy(i_idx_hbm.at[pl.ds(wid * nvec, nvec)], idx_i_v)

        iotas = [lax.iota(jnp.int32, 16) + 16 * c for c in range(D // 16)]

        def fire(tbl, emb_hbm, v, l):
            g0 = pl.multiple_of((v[l] // LT) * LT, LT)
            slot = tbl * RING + (l % RING)
            pltpu.make_async_copy(
                emb_hbm.at[:, pl.ds(g0, LT)], bufs[slot], sems[slot]).start()

        def extract(tbl, emb_hbm, rows_v, v, l, row):
            slot = tbl * RING + (l % RING)
            pltpu.make_async_copy(
                emb_hbm.at[:, pl.ds(0, LT)], bufs[slot], sems[slot]).wait()
            lane = jnp.broadcast_to(v[l] % LT, (16,))
            dst = rows_v.at[row]
            for c in range(D // 16):
                dst[pl.ds(16 * c, 16)] = plsc.load_gather(
                    bufs[slot], [iotas[c], lane])

        hvec = nvec // 4
        for half in range(4):
            def body(k, carry):
                pu, pi = carry
                vu = idx_u_v[half * hvec + k]
                vi = idx_i_v[half * hvec + k]
                for l in range(16):
                    e = k * 16 + l - RING
                    if l >= RING:
                        extract(0, u_embT_hbm, rows_u_v, vu, l - RING, e)
                        extract(1, i_embT_hbm, rows_i_v, vi, l - RING, e)
                    else:
                        @pl.when(k > 0)
                        def _():
                            extract(0, u_embT_hbm, rows_u_v, pu, l - RING, e)
                            extract(1, i_embT_hbm, rows_i_v, pi, l - RING, e)
                    fire(0, u_embT_hbm, vu, l)
                    fire(1, i_embT_hbm, vi, l)
                return vu, vi

            last_u, last_i = lax.fori_loop(
                0, hvec, body,
                (idx_u_v[half * hvec], idx_i_v[half * hvec]))
            for l in range(16 - RING, 16):
                e = (hvec - 1) * 16 + l
                extract(0, u_embT_hbm, rows_u_v, last_u, l, e)
                extract(1, i_embT_hbm, rows_i_v, last_i, l, e)

            out0 = base + half * (rows_per_w // 4)
            pltpu.sync_copy(rows_u_v, ue_hbm.at[pl.ds(out0, rows_per_w // 4)])
            pltpu.sync_copy(rows_i_v, ie_hbm.at[pl.ds(out0, rows_per_w // 4)])

    return gather


@functools.lru_cache(maxsize=None)
def _make_mlp(B: int, D: int, H: int, blk: int):
    def mlp(ue_ref, ie_ref, w1u_ref, w1i_ref, b1_ref, w2_ref, b2_ref, out_ref):
        h = jnp.dot(ue_ref[...], w1u_ref[...], preferred_element_type=jnp.float32)
        h = h + jnp.dot(ie_ref[...], w1i_ref[...], preferred_element_type=jnp.float32)
        h = jnp.maximum(h + b1_ref[...], 0.0)
        z = jnp.sum(h * w2_ref[...], axis=1) + b2_ref[0]
        out_ref[...] = jax.nn.sigmoid(z)

    return pl.pallas_call(
        mlp,
        grid=(B // blk,),
        in_specs=[
            pl.BlockSpec((blk, D), lambda b: (b, 0)),
            pl.BlockSpec((blk, D), lambda b: (b, 0)),
            pl.BlockSpec((D, H), lambda b: (0, 0)),
            pl.BlockSpec((D, H), lambda b: (0, 0)),
            pl.BlockSpec((1, H), lambda b: (0, 0)),
            pl.BlockSpec((1, H), lambda b: (0, 0)),
            pl.BlockSpec(memory_space=pltpu.SMEM),
        ],
        out_specs=pl.BlockSpec((blk,), lambda b: (b,)),
        out_shape=jax.ShapeDtypeStruct((B,), jnp.float32),
    )


def kernel(u, i, u_emb, i_emb, W1, b1, W2, b2):
    B = u.shape[0]
    V, D = u_emb.shape
    H = W1.shape[0]
    u_embT = jnp.swapaxes(u_emb, 0, 1)
    i_embT = jnp.swapaxes(i_emb, 0, 1)
    ue, ie = _make_gather(B, D, V)(
        u.astype(jnp.int32).reshape(B // 16, 16),
        i.astype(jnp.int32).reshape(B // 16, 16), u_embT, i_embT)
    w1u = W1[:, :D].T
    w1i = W1[:, D:].T
    return _make_mlp(B, D, H, 2048)(
        ue, ie, w1u, w1i, b1.reshape(1, H), W2, b2)


# speedup vs baseline: 2.6730x; 1.0127x over previous
import functools

import jax
import jax.numpy as jnp
from jax import lax
from jax.experimental import pallas as pl
from jax.experimental.pallas import tpu as pltpu
from jax.experimental.pallas import tpu_sc as plsc

NC = 2
NS = 16
NW = NC * NS
LT = 128
RING = 8


@functools.lru_cache(maxsize=None)
def _make_gather(B: int, D: int, V: int):
    rows_per_w = B // NW
    nvec = rows_per_w // 16
    mesh = plsc.VectorSubcoreMesh(
        core_axis_name="c", subcore_axis_name="s",
        num_cores=NC, num_subcores=NS)

    grp = [pltpu.VMEM((D, LT), jnp.float32)] * RING
    sems = [pltpu.SemaphoreType.DMA] * RING

    @functools.partial(
        pl.kernel,
        out_type=(jax.ShapeDtypeStruct((B, D), jnp.float32),
                  jax.ShapeDtypeStruct((B, D), jnp.float32)),
        mesh=mesh,
        scratch_types=[
            pltpu.VMEM((nvec, 16), jnp.int32),
            pltpu.VMEM((nvec, 16), jnp.int32),
            pltpu.VMEM((rows_per_w // 4, D), jnp.float32),
            *grp,
            *sems,
        ],
        compiler_params=pltpu.CompilerParams(needs_layout_passes=False),
    )
    def gather(u_idx_hbm, i_idx_hbm, u_embT_hbm, i_embT_hbm, ue_hbm, ie_hbm,
               idx_u_v, idx_i_v, rows_v, *ring):
        bufs = ring[:RING]
        sems = ring[RING:]
        wid = lax.axis_index("s") * NC + lax.axis_index("c")
        base = wid * rows_per_w
        pltpu.sync_copy(u_idx_hbm.at[pl.ds(wid * nvec, nvec)], idx_u_v)
        pltpu.sync_copy(i_idx_hbm.at[pl.ds(wid * nvec, nvec)], idx_i_v)

        iotas = [lax.iota(jnp.int32, 16) + 16 * c for c in range(D // 16)]

        def fire(emb_hbm, v, l):
            g0 = pl.multiple_of((v[l] // LT) * LT, LT)
            slot = l % RING
            pltpu.make_async_copy(
                emb_hbm.at[:, pl.ds(g0, LT)], bufs[slot], sems[slot]).start()

        def extract(emb_hbm, v, l, row):
            slot = l % RING
            pltpu.make_async_copy(
                emb_hbm.at[:, pl.ds(0, LT)], bufs[slot], sems[slot]).wait()
            lane = jnp.broadcast_to(v[l] % LT, (16,))
            dst = rows_v.at[row]
            for c in range(D // 16):
                dst[pl.ds(16 * c, 16)] = plsc.load_gather(
                    bufs[slot], [iotas[c], lane])

        hvec = nvec // 4
        for idx_v, emb_hbm, out_hbm in (
                (idx_u_v, u_embT_hbm, ue_hbm), (idx_i_v, i_embT_hbm, ie_hbm)):
            for half in range(4):
                def body(k, pv):
                    v = idx_v[half * hvec + k]
                    for l in range(16):
                        e = k * 16 + l - RING
                        if l >= RING:
                            extract(emb_hbm, v, l - RING, e)
                        else:
                            @pl.when(k > 0)
                            def _():
                                extract(emb_hbm, pv, l - RING, e)
                        fire(emb_hbm, v, l)
                    return v

                last_v = lax.fori_loop(0, hvec, body, idx_v[half * hvec])
                for l in range(16 - RING, 16):
                    e = (hvec - 1) * 16 + l
                    extract(emb_hbm, last_v, l, e)

                out0 = base + half * (rows_per_w // 4)
                pltpu.sync_copy(
                    rows_v, out_hbm.at[pl.ds(out0, rows_per_w // 4)])

    return gather


@functools.lru_cache(maxsize=None)
def _make_mlp(B: int, D: int, H: int, blk: int):
    def mlp(ue_ref, ie_ref, w1u_ref, w1i_ref, b1_ref, w2_ref, b2_ref, out_ref):
        h = jnp.dot(ue_ref[...], w1u_ref[...], preferred_element_type=jnp.float32)
        h = h + jnp.dot(ie_ref[...], w1i_ref[...], preferred_element_type=jnp.float32)
        h = jnp.maximum(h + b1_ref[...], 0.0)
        z = jnp.sum(h * w2_ref[...], axis=1) + b2_ref[0]
        out_ref[...] = jax.nn.sigmoid(z)

    return pl.pallas_call(
        mlp,
        grid=(B // blk,),
        in_specs=[
            pl.BlockSpec((blk, D), lambda b: (b, 0)),
            pl.BlockSpec((blk, D), lambda b: (b, 0)),
            pl.BlockSpec((D, H), lambda b: (0, 0)),
            pl.BlockSpec((D, H), lambda b: (0, 0)),
            pl.BlockSpec((1, H), lambda b: (0, 0)),
            pl.BlockSpec((1, H), lambda b: (0, 0)),
            pl.BlockSpec(memory_space=pltpu.SMEM),
        ],
        out_specs=pl.BlockSpec((blk,), lambda b: (b,)),
        out_shape=jax.ShapeDtypeStruct((B,), jnp.float32),
    )


def kernel(u, i, u_emb, i_emb, W1, b1, W2, b2):
    B = u.shape[0]
    V, D = u_emb.shape
    H = W1.shape[0]
    u_embT = jnp.swapaxes(u_emb, 0, 1)
    i_embT = jnp.swapaxes(i_emb, 0, 1)
    ue, ie = _make_gather(B, D, V)(
        u.astype(jnp.int32).reshape(B // 16, 16),
        i.astype(jnp.int32).reshape(B // 16, 16), u_embT, i_embT)
    w1u = W1[:, :D].T
    w1i = W1[:, D:].T
    return _make_mlp(B, D, H, 2048)(
        ue, ie, w1u, w1i, b1.reshape(1, H), W2, b2)
